# baseline (speedup 1.0000x reference)
import functools

import jax
import jax.numpy as jnp
from jax import lax
from jax.experimental import pallas as pl
from jax.experimental.pallas import tpu as pltpu
from jax.experimental.pallas import tpu_sc as plsc

_E = 320000
_D = 128
_N0, _N1, _N2 = 5, 6, 2
_NT = _N0 * _N1 * _N2

_NC = 2
_NS = 16
_NW = _NC * _NS
_PER_W = _E // _NW
_SB = 80
_GPC = 5
_CH = _SB * _GPC
_NCHUNK = _PER_W // _CH
_L = 16


@functools.cache
def _make_sc_kernel():
    @functools.partial(
        pl.kernel,
        mesh=plsc.VectorSubcoreMesh(core_axis_name="c", subcore_axis_name="s"),
        out_type=jax.ShapeDtypeStruct((_E, _D), jnp.float32),
        scratch_types=[
            pltpu.VMEM((13, _D), jnp.float32),
            pltpu.VMEM((_N1 * _N2, _D), jnp.float32),
            pltpu.VMEM((_NT, _D), jnp.float32),
            pltpu.VMEM((2, 3, _CH), jnp.int32),
            pltpu.VMEM((_GPC, _SB), jnp.int32),
            pltpu.VMEM((2, _CH, _D), jnp.float32),
            pltpu.VMEM_SHARED((_NT, _D), jnp.float32),
            pltpu.SemaphoreType.DMA,
            pltpu.SemaphoreType.DMA,
            pltpu.SemaphoreType.DMA,
        ],
    )
    def _sc_body(w0_hbm, w1_hbm, w2_hbm, ea_hbm, out_hbm,
                 wv, t12, tl, av, iv, rows_v, t_sh, gsem, ssem, asem):
        sid = lax.axis_index("s")
        wid = sid * _NC + lax.axis_index("c")
        base = wid * _PER_W
        @pl.when(sid == 0)
        def _():
            pltpu.sync_copy(w0_hbm, wv.at[pl.ds(0, _N0)])
            pltpu.sync_copy(w1_hbm, wv.at[pl.ds(_N0, _N1)])
            pltpu.sync_copy(w2_hbm, wv.at[pl.ds(_N0 + _N1, _N2)])

            def jk(m, carry):
                j = m // _N2
                k = m % _N2
                for h in range(_D // _L):
                    o = h * _L
                    t12[m, pl.ds(o, _L)] = (
                        wv[_N0 + j, pl.ds(o, _L)]
                        + wv[_N0 + _N1 + k, pl.ds(o, _L)]
                    )
                return carry

            lax.fori_loop(0, _N1 * _N2, jk, 0)

            def im(r, carry):
                i = r // (_N1 * _N2)
                m = r % (_N1 * _N2)
                for h in range(_D // _L):
                    o = h * _L
                    tl[r, pl.ds(o, _L)] = (
                        wv[i, pl.ds(o, _L)] + t12[m, pl.ds(o, _L)]
                    )
                return carry

            lax.fori_loop(0, _NT, im, 0)
            pltpu.sync_copy(tl, t_sh)

        plsc.subcore_barrier()

        def issue_attr(c, ab):
            return pltpu.async_copy(ea_hbm.at[wid, c], av.at[ab], asem)

        def wait_attr(c, ab):
            pltpu.make_async_copy(ea_hbm.at[wid, c], av.at[ab], asem).wait()

        def fold_from(ab):
            for g in range(_GPC):
                for k in range(_SB // _L):
                    o = g * _SB + k * _L
                    a0 = av[ab, 0, pl.ds(o, _L)]
                    a1 = av[ab, 1, pl.ds(o, _L)]
                    a2 = av[ab, 2, pl.ds(o, _L)]
                    iv[g, pl.ds(k * _L, _L)] = (
                        a0 * (_N1 * _N2) + a1 * _N2 + a2
                    )

        def gather_chunk(buf):
            return [
                pltpu.async_copy(
                    t_sh.at[iv.at[g]],
                    rows_v.at[buf].at[pl.ds(g * _SB, _SB)],
                    gsem,
                )
                for g in range(_GPC)
            ]

        def store_chunk(c, buf):
            return pltpu.async_copy(
                rows_v.at[buf], out_hbm.at[pl.ds(base + c * _CH, _CH)], ssem
            )

        issue_attr(0, 0).wait()
        fold_from(0)
        issue_attr(1, 1)
        for cp in gather_chunk(0):
            cp.wait()

        def chunk(c, carry):
            buf = lax.rem(c, 2)
            ab = lax.rem(c + 1, 2)
            st = store_chunk(c, buf)
            wait_attr(c + 1, ab)
            fold_from(ab)

            @pl.when(c + 2 < _NCHUNK)
            def _():
                issue_attr(c + 2, 1 - ab)

            for cp in gather_chunk(1 - buf):
                cp.wait()
            st.wait()
            return carry

        lax.fori_loop(0, _NCHUNK - 1, chunk, 0)
        store_chunk(_NCHUNK - 1, (_NCHUNK - 1) % 2).wait()

    return _sc_body


def kernel(edge_attr, W0, W1, W2):
    eap = (edge_attr.astype(jnp.int32)
           .reshape(_NW, _NCHUNK, _CH, 3)
           .transpose(0, 1, 3, 2))
    return _make_sc_kernel()(W0, W1, W2, eap)


# speedup vs baseline: 1.0288x; 1.0288x over previous
import functools

import jax
import jax.numpy as jnp
from jax import lax
from jax.experimental import pallas as pl
from jax.experimental.pallas import tpu as pltpu
from jax.experimental.pallas import tpu_sc as plsc

_E = 320000
_D = 128
_N0, _N1, _N2 = 5, 6, 2
_NT = _N0 * _N1 * _N2

_NC = 2
_NS = 16
_NW = _NC * _NS
_PER_W = _E // _NW
_SB = 80
_GPC = 5
_CH = _SB * _GPC
_NCHUNK = _PER_W // _CH
_L = 16


def _table_body(w0_ref, w1_ref, w2_ref, t_ref):
    r = lax.broadcasted_iota(jnp.int32, (_NT, 1), 0)
    oh0 = (lax.broadcasted_iota(jnp.int32, (_NT, _N0), 1)
           == r // (_N1 * _N2)).astype(jnp.float32)
    oh1 = (lax.broadcasted_iota(jnp.int32, (_NT, _N1), 1)
           == (r // _N2) % _N1).astype(jnp.float32)
    oh2 = (lax.broadcasted_iota(jnp.int32, (_NT, _N2), 1)
           == r % _N2).astype(jnp.float32)
    t_ref[...] = (
        jnp.dot(oh0, w0_ref[...], preferred_element_type=jnp.float32,
                precision=lax.Precision.HIGHEST)
        + jnp.dot(oh1, w1_ref[...], preferred_element_type=jnp.float32,
                  precision=lax.Precision.HIGHEST)
        + jnp.dot(oh2, w2_ref[...], preferred_element_type=jnp.float32,
                  precision=lax.Precision.HIGHEST)
    )


_build_table = pl.pallas_call(
    _table_body,
    out_shape=jax.ShapeDtypeStruct((_NT, _D), jnp.float32),
)


@functools.cache
def _make_sc_kernel():
    @functools.partial(
        pl.kernel,
        mesh=plsc.VectorSubcoreMesh(core_axis_name="c", subcore_axis_name="s"),
        out_type=jax.ShapeDtypeStruct((_E, _D), jnp.float32),
        scratch_types=[
            pltpu.VMEM((2, 3, _CH), jnp.int32),
            pltpu.VMEM((_GPC, _SB), jnp.int32),
            pltpu.VMEM((2, _CH, _D), jnp.float32),
            pltpu.VMEM_SHARED((_NT, _D), jnp.float32),
            pltpu.SemaphoreType.DMA,
            pltpu.SemaphoreType.DMA,
            pltpu.SemaphoreType.DMA,
        ],
    )
    def _sc_body(t_hbm, ea_hbm, out_hbm, av, iv, rows_v, t_sh,
                 gsem, ssem, asem):
        sid = lax.axis_index("s")
        wid = sid * _NC + lax.axis_index("c")
        base = wid * _PER_W
        @pl.when(sid == 0)
        def _():
            pltpu.sync_copy(t_hbm, t_sh)

        plsc.subcore_barrier()

        def issue_attr(c, ab):
            return pltpu.async_copy(ea_hbm.at[wid, c], av.at[ab], asem)

        def wait_attr(c, ab):
            pltpu.make_async_copy(ea_hbm.at[wid, c], av.at[ab], asem).wait()

        def fold_from(ab):
            for g in range(_GPC):
                for k in range(_SB // _L):
                    o = g * _SB + k * _L
                    a0 = av[ab, 0, pl.ds(o, _L)]
                    a1 = av[ab, 1, pl.ds(o, _L)]
                    a2 = av[ab, 2, pl.ds(o, _L)]
                    iv[g, pl.ds(k * _L, _L)] = (
                        a0 * (_N1 * _N2) + a1 * _N2 + a2
                    )

        def gather_chunk(buf):
            return [
                pltpu.async_copy(
                    t_sh.at[iv.at[g]],
                    rows_v.at[buf].at[pl.ds(g * _SB, _SB)],
                    gsem,
                )
                for g in range(_GPC)
            ]

        def store_chunk(c, buf):
            return pltpu.async_copy(
                rows_v.at[buf], out_hbm.at[pl.ds(base + c * _CH, _CH)], ssem
            )

        issue_attr(0, 0).wait()
        fold_from(0)
        issue_attr(1, 1)
        for cp in gather_chunk(0):
            cp.wait()

        def chunk(c, carry):
            buf = lax.rem(c, 2)
            ab = lax.rem(c + 1, 2)
            st = store_chunk(c, buf)
            wait_attr(c + 1, ab)
            fold_from(ab)

            @pl.when(c + 2 < _NCHUNK)
            def _():
                issue_attr(c + 2, 1 - ab)

            for cp in gather_chunk(1 - buf):
                cp.wait()
            st.wait()
            return carry

        lax.fori_loop(0, _NCHUNK - 1, chunk, 0)
        store_chunk(_NCHUNK - 1, (_NCHUNK - 1) % 2).wait()

    return _sc_body


def kernel(edge_attr, W0, W1, W2):
    eap = (edge_attr.astype(jnp.int32)
           .reshape(_NW, _NCHUNK, _CH, 3)
           .transpose(0, 1, 3, 2))
    t = _build_table(W0, W1, W2)
    return _make_sc_kernel()(t, eap)
